# Initial kernel scaffold; baseline (speedup 1.0000x reference)
import jax
import jax.numpy as jnp
from jax.experimental import pallas as pl


def kernel(x_drug, x_protein, x_sideeffect, edge_drug2drug, edge_drug2protein, edge_protein2drug, edge_protein2protein, edge_sideeffect2drug, params):
    raise NotImplementedError("write your pallas kernel here")


# speedup vs baseline: 33.9387x; 33.9387x over previous
import functools

import jax
import jax.numpy as jnp
from jax import lax
from jax.experimental import pallas as pl
from jax.experimental.pallas import tpu as pltpu
from jax.experimental.pallas import tpu_sc as plsc

HID = 128
HEADS = 8
OUTF = 16
NNODE = 10000
E = 320000
ROWW = 144
ERW = 16
NC, NS, LANES = 2, 16, 16
CHUNK = 80
EPT = E // (NC * NS)
NCHUNK = EPT // CHUNK
ZCH = NNODE // CHUNK
NREL = 5

_RELS = (('drug2drug', 'drug', 'drug'),
         ('drug2protein', 'drug', 'protein'),
         ('protein2drug', 'protein', 'drug'),
         ('protein2protein', 'protein', 'protein'),
         ('sideeffect2drug', 'sideeffect', 'drug'))


def _i16(v):
    return jnp.full((LANES,), v, dtype=jnp.int32)


def _edge_body(*refs):
    zels = refs[0:5]
    erts = refs[5:10]
    srcs = refs[10:15]
    dsts = refs[15:20]
    out = refs[20]
    srcv, dstv, zbuf, erbuf, zerobuf, accum, sem1, sem2 = refs[21:]

    cid = lax.axis_index("c")
    sid = lax.axis_index("s")
    base_edge = (cid * NS + sid) * EPT

    zvec = jnp.zeros((LANES,), jnp.float32)

    def _zrow(c, _):
        for j in range(ROWW // LANES):
            zerobuf[c, pl.ds(j * LANES, LANES)] = zvec
        return 0

    lax.fori_loop(0, CHUNK, _zrow, 0)

    iota = lax.iota(jnp.int32, LANES)

    for r in range(NREL):
        for k in range(ZCH // NS + 1):
            ch = sid + NS * k
            if k * NS < ZCH:
                @pl.when(ch < ZCH)
                def _():
                    pltpu.sync_copy(zerobuf, accum.at[pl.ds(ch * CHUNK, CHUNK)])
        plsc.subcore_barrier()

        def _chunk(i, _):
            off = base_edge + i * CHUNK
            pltpu.sync_copy(srcs[r].at[pl.ds(off, CHUNK)], srcv)
            pltpu.sync_copy(dsts[r].at[pl.ds(off, CHUNK)], dstv)
            g1 = pltpu.async_copy(zels[r].at[srcv], zbuf, sem1)
            g2 = pltpu.async_copy(erts[r].at[dstv], erbuf, sem2)
            g1.wait()
            g2.wait()

            def _group(g, _):
                rows = iota + g * LANES
                for h in range(HEADS):
                    elv = plsc.load_gather(zbuf, [rows, _i16(HID + h)])
                    erv = plsc.load_gather(erbuf, [rows, _i16(h)])
                    t = elv + erv
                    s = jnp.exp(jnp.maximum(t, t * 0.2))
                    plsc.store_scatter(zbuf, [rows, _i16(HID + h)], s)
                    for f in range(OUTF):
                        col = _i16(h * OUTF + f)
                        zc = plsc.load_gather(zbuf, [rows, col])
                        plsc.store_scatter(zbuf, [rows, col], zc * s)
                return 0

            lax.fori_loop(0, CHUNK // LANES, _group, 0)
            pltpu.sync_copy(zbuf, accum.at[dstv], add=True)
            return 0

        lax.fori_loop(0, NCHUNK, _chunk, 0)
        plsc.subcore_barrier()

        for k in range(ZCH // NS + 1):
            ch = sid + NS * k
            if k * NS < ZCH:
                @pl.when(ch < ZCH)
                def _():
                    pltpu.sync_copy(accum.at[pl.ds(ch * CHUNK, CHUNK)],
                                    out.at[cid, r, pl.ds(ch * CHUNK, CHUNK)])
        plsc.subcore_barrier()


_edge_call = functools.partial(
    pl.kernel,
    out_type=jax.ShapeDtypeStruct((NC, NREL, NNODE, ROWW), jnp.float32),
    mesh=plsc.VectorSubcoreMesh(core_axis_name="c", subcore_axis_name="s"),
    scratch_types=[
        pltpu.VMEM((CHUNK,), jnp.int32),
        pltpu.VMEM((CHUNK,), jnp.int32),
        pltpu.VMEM((CHUNK, ROWW), jnp.float32),
        pltpu.VMEM((CHUNK, ERW), jnp.float32),
        pltpu.VMEM((CHUNK, ROWW), jnp.float32),
        pltpu.VMEM_SHARED((NNODE, ROWW), jnp.float32),
        pltpu.SemaphoreType.DMA,
        pltpu.SemaphoreType.DMA,
    ],
    compiler_params=pltpu.CompilerParams(use_tc_tiling_on_sc=False,
                                         needs_layout_passes=False),
)(_edge_body)


def _fold(a):
    return (a[:, :, None] * jnp.eye(HEADS, dtype=a.dtype)[:, None, :]).reshape(HID, HEADS)


def kernel(x_drug, x_protein, x_sideeffect, edge_drug2drug, edge_drug2protein,
           edge_protein2drug, edge_protein2protein, edge_sideeffect2drug, params):
    p = params
    xs = {'drug': x_drug, 'protein': x_protein, 'sideeffect': x_sideeffect}
    edges = {'drug2drug': edge_drug2drug, 'drug2protein': edge_drug2protein,
             'protein2drug': edge_protein2drug, 'protein2protein': edge_protein2protein,
             'sideeffect2drug': edge_sideeffect2drug}

    raw = {nt: xs[nt] @ p['proj_W_' + nt] + p['proj_b_' + nt] for nt in xs}
    h = raw
    zpad = jnp.zeros((NNODE, 8), jnp.float32)
    for l in range(2):
        zels, erts, srcs, dsts = [], [], [], []
        for rel, s, d in _RELS:
            W = p['gat_W_%d_%s' % (l, rel)]
            z = h[s] @ W
            el = z @ _fold(p['gat_al_%d_%s' % (l, rel)])
            zels.append(jnp.concatenate([z, el, zpad], axis=1))
            er = h[d] @ (W @ _fold(p['gat_ar_%d_%s' % (l, rel)]))
            erts.append(jnp.concatenate([er, zpad], axis=1))
            e = edges[rel]
            srcs.append(e[0].astype(jnp.int32))
            dsts.append(e[1].astype(jnp.int32))

        acc = _edge_call(*zels, *erts, *srcs, *dsts)
        acc = acc[0] + acc[1]

        outs = {}
        for r, (rel, s, d) in enumerate(_RELS):
            num = acc[r, :, :HID].reshape(NNODE, HEADS, OUTF)
            den = acc[r, :, HID:HID + HEADS]
            b = p['gat_b_%d_%s' % (l, rel)].reshape(1, HEADS, OUTF)
            o = num / (den[:, :, None] + 1e-9) + b
            outs.setdefault(d, []).append(o.reshape(NNODE, HID))

        newh = {}
        for d, feats in outs.items():
            W1, b1 = p['sem_W1_%d' % l], p['sem_b1_%d' % l]
            W2, b2 = p['sem_W2_%d' % l], p['sem_b2_%d' % l]
            ws = [jnp.tanh(f @ W1 + b1) @ W2 + b2 for f in feats]
            m = functools.reduce(jnp.maximum, ws)
            es = [jnp.exp(w - m) for w in ws]
            dn = sum(es)
            newh[d] = sum((e_ / dn) * f for e_, f in zip(es, feats))
        h = dict(raw)
        h.update(newh)

    return h['drug'], h['protein'], h['sideeffect']


# speedup vs baseline: 41.9324x; 1.2355x over previous
import functools

import jax
import jax.numpy as jnp
from jax import lax
from jax.experimental import pallas as pl
from jax.experimental.pallas import tpu as pltpu
from jax.experimental.pallas import tpu_sc as plsc

HID = 128
HEADS = 8
OUTF = 16
NNODE = 10000
E = 320000
ROWW = 144
ERW = 16
NC, NS, LANES = 2, 16, 16
CHUNK = 80
EPT = E // (NC * NS)
NCHUNK = EPT // CHUNK
ZCH = NNODE // CHUNK
NREL = 5

_RELS = (('drug2drug', 'drug', 'drug'),
         ('drug2protein', 'drug', 'protein'),
         ('protein2drug', 'protein', 'drug'),
         ('protein2protein', 'protein', 'protein'),
         ('sideeffect2drug', 'sideeffect', 'drug'))


def _i16(v):
    return jnp.full((LANES,), v, dtype=jnp.int32)


def _edge_body(*refs):
    zels = refs[0:5]
    erts = refs[5:10]
    e3s = refs[10:15]
    out = refs[15]
    (islots, zb0, zb1, ob, eb0, eb1, accum,
     gs0, gs1, es0, es1, ss, is0, is1, is2, is3) = refs[16:]
    isems = (is0, is1, is2, is3)
    zbs = (zb0, zb1)
    ebs = (eb0, eb1)
    gss = (gs0, gs1)
    ess = (es0, es1)

    cid = lax.axis_index("c")
    sid = lax.axis_index("s")
    tid = cid * NS + sid
    cbase = tid * NCHUNK

    zvec = jnp.zeros((LANES,), jnp.float32)
    iota = lax.iota(jnp.int32, LANES)

    def _zero_ob(c, _):
        for j in range(ROWW // LANES):
            ob[c, pl.ds(j * LANES, LANES)] = zvec
        return 0

    def _compute(zbuf, erbuf):
        def _group(g, _):
            rows = iota + g * LANES

            def _head(h, _):
                hv = jnp.full((LANES,), 0, jnp.int32) + h
                elv = plsc.load_gather(zbuf, [rows, hv + HID])
                erv = plsc.load_gather(erbuf, [rows, hv])
                t = elv + erv
                s = jnp.exp(jnp.maximum(t, t * 0.2))
                plsc.store_scatter(ob, [rows, hv + HID], s)
                base = hv * OUTF
                for f in range(OUTF):
                    col = base + f
                    zc = plsc.load_gather(zbuf, [rows, col])
                    plsc.store_scatter(ob, [rows, col], zc * s)
                return 0

            lax.fori_loop(0, HEADS, _head, 0)
            return 0

        lax.fori_loop(0, CHUNK // LANES, _group, 0)

    for r in range(NREL):
        zel, ert, e3 = zels[r], erts[r], e3s[r]

        def _icopy(c, slot):
            pltpu.async_copy(e3.at[c], islots.at[slot], isems[slot])

        def _iwait(slot):
            pltpu.make_async_copy(e3.at[0], islots.at[slot],
                                  isems[slot]).wait()

        def _gissue(c, slot, b):
            pltpu.async_copy(zel.at[islots.at[slot, 0]], zbs[b], gss[b])
            pltpu.async_copy(ert.at[islots.at[slot, 1]], ebs[b], ess[b])

        def _gwait(b):
            pltpu.make_async_copy(zel.at[islots.at[0, 0]], zbs[b],
                                  gss[b]).wait()
            pltpu.make_async_copy(ert.at[islots.at[0, 1]], ebs[b],
                                  ess[b]).wait()

        def _sissue(slot):
            pltpu.async_copy(ob, accum.at[islots.at[slot, 1]], ss, add=True)

        def _swait():
            pltpu.make_async_copy(ob, accum.at[islots.at[0, 1]], ss).wait()

        lax.fori_loop(0, CHUNK, _zero_ob, 0)
        for k in range(ZCH // NS + 1):
            ch = sid + NS * k
            if k * NS < ZCH:
                @pl.when(ch < ZCH)
                def _():
                    pltpu.sync_copy(ob, accum.at[pl.ds(ch * CHUNK, CHUNK)])
        plsc.subcore_barrier()

        for s in range(3):
            _icopy(cbase + s, s)
        _iwait(0)
        _gissue(cbase, 0, 0)
        _iwait(1)
        _gissue(cbase + 1, 1, 1)

        def _quad(k, _):
            for j in range(4):
                c = 4 * k + j
                b = j % 2
                _gwait(b)
                if j == 0:
                    @pl.when(k > 0)
                    def _():
                        _swait()
                else:
                    _swait()
                if 4 * 30 + j + 3 < NCHUNK:
                    _icopy(cbase + c + 3, (j + 3) % 4)
                else:
                    @pl.when(c + 3 < NCHUNK)
                    def _():
                        _icopy(cbase + c + 3, (j + 3) % 4)
                _compute(zbs[b], ebs[b])
                _sissue(j)
                if 4 * 30 + j + 2 < NCHUNK:
                    _iwait((j + 2) % 4)
                    _gissue(cbase + c + 2, (j + 2) % 4, b)
                else:
                    @pl.when(c + 2 < NCHUNK)
                    def _():
                        _iwait((j + 2) % 4)
                        _gissue(cbase + c + 2, (j + 2) % 4, b)
            return 0

        lax.fori_loop(0, NCHUNK // 4, _quad, 0)

        _gwait(0)
        _swait()
        _compute(zb0, eb0)
        _sissue(0)
        _swait()
        plsc.subcore_barrier()

        for k in range(ZCH // NS + 1):
            ch = sid + NS * k
            if k * NS < ZCH:
                @pl.when(ch < ZCH)
                def _():
                    pltpu.sync_copy(accum.at[pl.ds(ch * CHUNK, CHUNK)],
                                    out.at[cid, r, pl.ds(ch * CHUNK, CHUNK)])
        plsc.subcore_barrier()


_edge_call = functools.partial(
    pl.kernel,
    out_type=jax.ShapeDtypeStruct((NC, NREL, NNODE, ROWW), jnp.float32),
    mesh=plsc.VectorSubcoreMesh(core_axis_name="c", subcore_axis_name="s"),
    scratch_types=[
        pltpu.VMEM((4, 2, CHUNK), jnp.int32),
        pltpu.VMEM((CHUNK, ROWW), jnp.float32),
        pltpu.VMEM((CHUNK, ROWW), jnp.float32),
        pltpu.VMEM((CHUNK, ROWW), jnp.float32),
        pltpu.VMEM((CHUNK, ERW), jnp.float32),
        pltpu.VMEM((CHUNK, ERW), jnp.float32),
        pltpu.VMEM_SHARED((NNODE, ROWW), jnp.float32),
        pltpu.SemaphoreType.DMA,
        pltpu.SemaphoreType.DMA,
        pltpu.SemaphoreType.DMA,
        pltpu.SemaphoreType.DMA,
        pltpu.SemaphoreType.DMA,
        pltpu.SemaphoreType.DMA,
        pltpu.SemaphoreType.DMA,
        pltpu.SemaphoreType.DMA,
        pltpu.SemaphoreType.DMA,
    ],
    compiler_params=pltpu.CompilerParams(use_tc_tiling_on_sc=False,
                                         needs_layout_passes=False),
)(_edge_body)


def _fold(a):
    return (a[:, :, None] * jnp.eye(HEADS, dtype=a.dtype)[:, None, :]).reshape(HID, HEADS)


def kernel(x_drug, x_protein, x_sideeffect, edge_drug2drug, edge_drug2protein,
           edge_protein2drug, edge_protein2protein, edge_sideeffect2drug, params):
    p = params
    xs = {'drug': x_drug, 'protein': x_protein, 'sideeffect': x_sideeffect}
    edges = {'drug2drug': edge_drug2drug, 'drug2protein': edge_drug2protein,
             'protein2drug': edge_protein2drug, 'protein2protein': edge_protein2protein,
             'sideeffect2drug': edge_sideeffect2drug}

    raw = {nt: xs[nt] @ p['proj_W_' + nt] + p['proj_b_' + nt] for nt in xs}
    h = raw
    zpad = jnp.zeros((NNODE, 8), jnp.float32)
    for l in range(2):
        zels, erts, srcs = [], [], []
        for rel, s, d in _RELS:
            W = p['gat_W_%d_%s' % (l, rel)]
            z = h[s] @ W
            el = z @ _fold(p['gat_al_%d_%s' % (l, rel)])
            zels.append(jnp.concatenate([z, el, zpad], axis=1))
            er = h[d] @ (W @ _fold(p['gat_ar_%d_%s' % (l, rel)]))
            erts.append(jnp.concatenate([er, zpad], axis=1))
            e3 = edges[rel].astype(jnp.int32).reshape(2, E // CHUNK, CHUNK)
            srcs.append(e3.transpose(1, 0, 2))

        acc = _edge_call(*zels, *erts, *srcs)
        acc = acc[0] + acc[1]

        outs = {}
        for r, (rel, s, d) in enumerate(_RELS):
            num = acc[r, :, :HID].reshape(NNODE, HEADS, OUTF)
            den = acc[r, :, HID:HID + HEADS]
            b = p['gat_b_%d_%s' % (l, rel)].reshape(1, HEADS, OUTF)
            o = num / (den[:, :, None] + 1e-9) + b
            outs.setdefault(d, []).append(o.reshape(NNODE, HID))

        newh = {}
        for d, feats in outs.items():
            W1, b1 = p['sem_W1_%d' % l], p['sem_b1_%d' % l]
            W2, b2 = p['sem_W2_%d' % l], p['sem_b2_%d' % l]
            ws = [jnp.tanh(f @ W1 + b1) @ W2 + b2 for f in feats]
            m = functools.reduce(jnp.maximum, ws)
            es = [jnp.exp(w - m) for w in ws]
            dn = sum(es)
            newh[d] = sum((e_ / dn) * f for e_, f in zip(es, feats))
        h = dict(raw)
        h.update(newh)

    return h['drug'], h['protein'], h['sideeffect']


# speedup vs baseline: 43.9871x; 1.0490x over previous
import functools

import jax
import jax.numpy as jnp
from jax import lax
from jax.experimental import pallas as pl
from jax.experimental.pallas import tpu as pltpu
from jax.experimental.pallas import tpu_sc as plsc

HID = 128
HEADS = 8
OUTF = 16
NNODE = 10000
E = 320000
ROWW = 144
ERW = 16
NC, NS, LANES = 2, 16, 16
CHUNK = 80
EPT = E // (NC * NS)
NCHUNK = EPT // CHUNK
ZCH = NNODE // CHUNK
NREL = 5

_RELS = (('drug2drug', 'drug', 'drug'),
         ('drug2protein', 'drug', 'protein'),
         ('protein2drug', 'protein', 'drug'),
         ('protein2protein', 'protein', 'protein'),
         ('sideeffect2drug', 'sideeffect', 'drug'))


def _i16(v):
    return jnp.full((LANES,), v, dtype=jnp.int32)


def _edge_body(*refs):
    zels = refs[0:5]
    erts = refs[5:10]
    e3s = refs[10:15]
    out = refs[15]
    (islots, zb0, zb1, ob, eb0, eb1, accum,
     gs0, gs1, es0, es1, ss, is0, is1, is2, is3) = refs[16:]
    isems = (is0, is1, is2, is3)
    zbs = (zb0, zb1)
    ebs = (eb0, eb1)
    gss = (gs0, gs1)
    ess = (es0, es1)

    cid = lax.axis_index("c")
    sid = lax.axis_index("s")
    tid = cid * NS + sid
    cbase = tid * NCHUNK

    zvec = jnp.zeros((LANES,), jnp.float32)
    iota = lax.iota(jnp.int32, LANES)

    def _zero_ob(c, _):
        for j in range(ROWW // LANES):
            ob[c, pl.ds(j * LANES, LANES)] = zvec
        return 0

    def _compute(zbuf, erbuf):
        def _group(g, _):
            rows = iota + g * LANES

            def _head(h, _):
                hv = jnp.full((LANES,), 0, jnp.int32) + h
                elv = plsc.load_gather(zbuf, [rows, hv + HID])
                erv = plsc.load_gather(erbuf, [rows, hv])
                t = elv + erv
                s = jnp.exp(jnp.maximum(t, t * 0.2))
                plsc.store_scatter(ob, [rows, hv + HID], s)
                base = hv * OUTF
                for f in range(OUTF):
                    col = base + f
                    zc = plsc.load_gather(zbuf, [rows, col])
                    plsc.store_scatter(ob, [rows, col], zc * s)
                return 0

            lax.fori_loop(0, HEADS, _head, 0)
            return 0

        lax.fori_loop(0, CHUNK // LANES, _group, 0)

    for r in range(NREL):
        zel, ert, e3 = zels[r], erts[r], e3s[r]

        def _icopy(c, slot):
            pltpu.async_copy(e3.at[c], islots.at[slot], isems[slot])

        def _iwait(slot):
            pltpu.make_async_copy(e3.at[0], islots.at[slot],
                                  isems[slot]).wait()

        def _gissue(c, slot, b):
            pltpu.async_copy(zel.at[islots.at[slot, 0]], zbs[b], gss[b])
            pltpu.async_copy(ert.at[islots.at[slot, 1]], ebs[b], ess[b])

        def _gwait(b):
            pltpu.make_async_copy(zel.at[islots.at[0, 0]], zbs[b],
                                  gss[b]).wait()
            pltpu.make_async_copy(ert.at[islots.at[0, 1]], ebs[b],
                                  ess[b]).wait()

        def _sissue(slot):
            pltpu.async_copy(ob, accum.at[islots.at[slot, 1]], ss, add=True)

        def _swait():
            pltpu.make_async_copy(ob, accum.at[islots.at[0, 1]], ss).wait()

        lax.fori_loop(0, CHUNK, _zero_ob, 0)
        for k in range(ZCH // NS + 1):
            ch = sid + NS * k
            if k * NS < ZCH:
                @pl.when(ch < ZCH)
                def _():
                    pltpu.sync_copy(ob, accum.at[pl.ds(ch * CHUNK, CHUNK)])
        plsc.subcore_barrier()

        for s in range(3):
            _icopy(cbase + s, s)
        _iwait(0)
        _gissue(cbase, 0, 0)
        _iwait(1)
        _gissue(cbase + 1, 1, 1)

        def _quad(k, _):
            for j in range(4):
                c = 4 * k + j
                b = j % 2
                _gwait(b)
                if j == 0:
                    @pl.when(k > 0)
                    def _():
                        _swait()
                else:
                    _swait()
                if 4 * 30 + j + 3 < NCHUNK:
                    _icopy(cbase + c + 3, (j + 3) % 4)
                else:
                    @pl.when(c + 3 < NCHUNK)
                    def _():
                        _icopy(cbase + c + 3, (j + 3) % 4)
                _compute(zbs[b], ebs[b])
                _sissue(j)
                if 4 * 30 + j + 2 < NCHUNK:
                    _iwait((j + 2) % 4)
                    _gissue(cbase + c + 2, (j + 2) % 4, b)
                else:
                    @pl.when(c + 2 < NCHUNK)
                    def _():
                        _iwait((j + 2) % 4)
                        _gissue(cbase + c + 2, (j + 2) % 4, b)
            return 0

        lax.fori_loop(0, NCHUNK // 4, _quad, 0)

        _gwait(0)
        _swait()
        _compute(zb0, eb0)
        _sissue(0)
        _swait()
        plsc.subcore_barrier()

        for k in range(ZCH // NS + 1):
            ch = sid + NS * k
            if k * NS < ZCH:
                @pl.when(ch < ZCH)
                def _():
                    pltpu.sync_copy(accum.at[pl.ds(ch * CHUNK, CHUNK)],
                                    out.at[cid, r, pl.ds(ch * CHUNK, CHUNK)])
        plsc.subcore_barrier()


_edge_call = functools.partial(
    pl.kernel,
    out_type=jax.ShapeDtypeStruct((NC, NREL, NNODE, ROWW), jnp.float32),
    mesh=plsc.VectorSubcoreMesh(core_axis_name="c", subcore_axis_name="s"),
    scratch_types=[
        pltpu.VMEM((4, 2, CHUNK), jnp.int32),
        pltpu.VMEM((CHUNK, ROWW), jnp.float32),
        pltpu.VMEM((CHUNK, ROWW), jnp.float32),
        pltpu.VMEM((CHUNK, ROWW), jnp.float32),
        pltpu.VMEM((CHUNK, ERW), jnp.float32),
        pltpu.VMEM((CHUNK, ERW), jnp.float32),
        pltpu.VMEM_SHARED((NNODE, ROWW), jnp.float32),
        pltpu.SemaphoreType.DMA,
        pltpu.SemaphoreType.DMA,
        pltpu.SemaphoreType.DMA,
        pltpu.SemaphoreType.DMA,
        pltpu.SemaphoreType.DMA,
        pltpu.SemaphoreType.DMA,
        pltpu.SemaphoreType.DMA,
        pltpu.SemaphoreType.DMA,
        pltpu.SemaphoreType.DMA,
    ],
    compiler_params=pltpu.CompilerParams(use_tc_tiling_on_sc=False,
                                         needs_layout_passes=False),
)(_edge_body)


def _fold(a):
    return (a[:, :, None] * jnp.eye(HEADS, dtype=a.dtype)[:, None, :]).reshape(HID, HEADS)


_SRC = (0, 0, 1, 1, 2)
_DST = (0, 1, 0, 1, 0)
BLK = 1000
NBLK = NNODE // BLK
_F32 = jnp.float32


def _dot(a, b):
    return jnp.dot(a, b, preferred_element_type=_F32)


def _k1_body(*refs):
    xd, xp, xs, wp, bp, wc, we = refs[:7]
    rd, rp, rs = refs[7:10]
    zels = refs[10:15]
    erts = refs[15:20]
    raws = []
    for t, x in enumerate((xd, xp, xs)):
        raws.append(_dot(x[...], wp[t]) + bp[t][None, :])
    rd[...] = raws[0]
    rp[...] = raws[1]
    rs[...] = raws[2]
    for r in range(NREL):
        zels[r][...] = _dot(raws[_SRC[r]], wc[r])
        erts[r][...] = _dot(raws[_DST[r]], we[r])


def _semantic(acc, sel, gb, w1, b1, w2):
    feats = []
    for r in range(NREL):
        a = acc[0, r] + acc[1, r]
        den = _dot(a[:, HID:HID + HEADS], sel[...])
        feats.append(a[:, :HID] / (den + 1e-9) + gb[r][None, :])
    hs = {}
    for d, rels in ((0, (0, 2, 4)), (1, (1, 3))):
        ws = [_dot(jnp.tanh(_dot(feats[r], w1[...]) + b1[...]), w2[...])
              for r in rels]
        m = functools.reduce(jnp.maximum, ws)
        es = [jnp.exp(w - m) for w in ws]
        dn = sum(es)
        hs[d] = sum((e_ / dn) * feats[r] for e_, r in zip(es, rels))
    return hs


def _k2_body(*refs):
    acc, sel, gb, w1, b1, w2, rs_blk, wc, we = refs[:9]
    zels = refs[9:14]
    erts = refs[14:19]
    hs = _semantic(acc, sel, gb, w1, b1, w2)
    hmap = (hs[0], hs[1], rs_blk[...])
    for r in range(NREL):
        zels[r][...] = _dot(hmap[_SRC[r]], wc[r])
        erts[r][...] = _dot(hmap[_DST[r]], we[r])


def _k3_body(*refs):
    acc, sel, gb, w1, b1, w2, hd, hp = refs
    hs = _semantic(acc, sel, gb, w1, b1, w2)
    hd[...] = hs[0]
    hp[...] = hs[1]


_row_spec = pl.BlockSpec((BLK, HID), lambda i: (i, 0))
_zel_spec = pl.BlockSpec((BLK, ROWW), lambda i: (i, 0))
_ert_spec = pl.BlockSpec((BLK, ERW), lambda i: (i, 0))
_zel_shape = jax.ShapeDtypeStruct((NNODE, ROWW), _F32)
_ert_shape = jax.ShapeDtypeStruct((NNODE, ERW), _F32)
_row_shape = jax.ShapeDtypeStruct((NNODE, HID), _F32)


def _whole(*shape):
    return pl.BlockSpec(shape, lambda i: tuple(0 for _ in shape))


_acc_spec = pl.BlockSpec((NC, NREL, BLK, ROWW), lambda i: (0, 0, i, 0))
_sem_specs = [_whole(8, HID), _whole(NREL, HID), _whole(HID, HID),
              _whole(1, HID), _whole(HID, 1)]

_k1 = pl.pallas_call(
    _k1_body,
    grid=(NBLK,),
    in_specs=[_row_spec, _row_spec, _row_spec, _whole(3, HID, HID),
              _whole(3, HID), _whole(NREL, HID, ROWW), _whole(NREL, HID, ERW)],
    out_specs=[_row_spec] * 3 + [_zel_spec] * NREL + [_ert_spec] * NREL,
    out_shape=[_row_shape] * 3 + [_zel_shape] * NREL + [_ert_shape] * NREL,
)

_k2 = pl.pallas_call(
    _k2_body,
    grid=(NBLK,),
    in_specs=[_acc_spec] + _sem_specs
    + [_row_spec, _whole(NREL, HID, ROWW), _whole(NREL, HID, ERW)],
    out_specs=[_zel_spec] * NREL + [_ert_spec] * NREL,
    out_shape=[_zel_shape] * NREL + [_ert_shape] * NREL,
)

_k3 = pl.pallas_call(
    _k3_body,
    grid=(NBLK,),
    in_specs=[_acc_spec] + _sem_specs,
    out_specs=[_row_spec, _row_spec],
    out_shape=[_row_shape, _row_shape],
)


def kernel(x_drug, x_protein, x_sideeffect, edge_drug2drug, edge_drug2protein,
           edge_protein2drug, edge_protein2protein, edge_sideeffect2drug, params):
    p = params

    wp = jnp.stack([p['proj_W_drug'], p['proj_W_protein'], p['proj_W_sideeffect']])
    bp = jnp.stack([p['proj_b_drug'], p['proj_b_protein'], p['proj_b_sideeffect']])
    sel = (jnp.arange(HID, dtype=jnp.int32) // OUTF ==
           jnp.arange(HEADS, dtype=jnp.int32)[:, None]).astype(_F32)
    zpad8 = jnp.zeros((HID, 8), _F32)
    wcs, wes, gbs = [], [], []
    for l in range(2):
        wc_l, we_l, gb_l = [], [], []
        for rel, s, d in _RELS:
            W = p['gat_W_%d_%s' % (l, rel)]
            wc_l.append(jnp.concatenate(
                [W, W @ _fold(p['gat_al_%d_%s' % (l, rel)]), zpad8], axis=1))
            we_l.append(jnp.concatenate(
                [W @ _fold(p['gat_ar_%d_%s' % (l, rel)]), zpad8], axis=1))
            gb_l.append(p['gat_b_%d_%s' % (l, rel)])
        wcs.append(jnp.stack(wc_l))
        wes.append(jnp.stack(we_l))
        gbs.append(jnp.stack(gb_l))
    sems = [(sel, gbs[l], p['sem_W1_%d' % l], p['sem_b1_%d' % l].reshape(1, HID),
             p['sem_W2_%d' % l]) for l in range(2)]
    e3s = [edges.astype(jnp.int32).reshape(2, E // CHUNK, CHUNK).transpose(1, 0, 2)
           for edges in (edge_drug2drug, edge_drug2protein, edge_protein2drug,
                         edge_protein2protein, edge_sideeffect2drug)]

    outs1 = _k1(x_drug, x_protein, x_sideeffect, wp, bp, wcs[0], wes[0])
    raw_side = outs1[2]
    acc0 = _edge_call(*outs1[3:8], *outs1[8:13], *e3s)
    outs2 = _k2(acc0, *sems[0], raw_side, wcs[1], wes[1])
    acc1 = _edge_call(*outs2[0:5], *outs2[5:10], *e3s)
    h_drug, h_prot = _k3(acc1, *sems[1])
    return h_drug, h_prot, raw_side


# speedup vs baseline: 92.0955x; 2.0937x over previous
import functools

import jax
import jax.numpy as jnp
from jax import lax
from jax.experimental import pallas as pl
from jax.experimental.pallas import tpu as pltpu
from jax.experimental.pallas import tpu_sc as plsc

HID = 128
HEADS = 8
OUTF = 16
NNODE = 10000
E = 320000
ROWW = 144
ERW = 16
NC, NS, LANES = 2, 16, 16
CHUNK = 80
EPT = E // (NC * NS)
NCHUNK = EPT // CHUNK
ZCH = NNODE // CHUNK
NREL = 5

_RELS = (('drug2drug', 'drug', 'drug'),
         ('drug2protein', 'drug', 'protein'),
         ('protein2drug', 'protein', 'drug'),
         ('protein2protein', 'protein', 'protein'),
         ('sideeffect2drug', 'sideeffect', 'drug'))


def _i16(v):
    return jnp.full((LANES,), v, dtype=jnp.int32)


def _edge_body(*refs):
    zels = refs[0:5]
    erts = refs[5:10]
    e3s = refs[10:15]
    out = refs[15]
    (islots, zb0, zb1, ob, eb0, eb1, accum,
     gs0, gs1, es0, es1, ss, is0, is1, is2, is3) = refs[16:]
    isems = (is0, is1, is2, is3)
    zbs = (zb0, zb1)
    ebs = (eb0, eb1)
    gss = (gs0, gs1)
    ess = (es0, es1)

    cid = lax.axis_index("c")
    sid = lax.axis_index("s")
    tid = cid * NS + sid
    cbase = tid * NCHUNK

    zvec = jnp.zeros((LANES,), jnp.float32)
    iota = lax.iota(jnp.int32, LANES)

    def _zero_ob(c, _):
        for j in range(ROWW // LANES):
            ob[c, pl.ds(j * LANES, LANES)] = zvec
        return 0

    def _compute(zbuf, erbuf):
        def _group(g, _):
            rows = iota + g * LANES

            def _head(h, _):
                hv = jnp.full((LANES,), 0, jnp.int32) + h
                elv = plsc.load_gather(zbuf, [rows, hv + HID])
                erv = plsc.load_gather(erbuf, [rows, hv])
                t = elv + erv
                s = jnp.exp(jnp.maximum(t, t * 0.2))
                plsc.store_scatter(ob, [rows, hv + HID], s)
                base = hv * OUTF
                zcs = [plsc.load_gather(zbuf, [rows, base + f])
                       for f in range(OUTF)]
                for f in range(OUTF):
                    plsc.store_scatter(ob, [rows, base + f], zcs[f] * s)
                return 0

            lax.fori_loop(0, HEADS, _head, 0)
            return 0

        lax.fori_loop(0, CHUNK // LANES, _group, 0)

    for r in range(NREL):
        zel, ert, e3 = zels[r], erts[r], e3s[r]

        def _icopy(c, slot):
            pltpu.async_copy(e3.at[c], islots.at[slot], isems[slot])

        def _iwait(slot):
            pltpu.make_async_copy(e3.at[0], islots.at[slot],
                                  isems[slot]).wait()

        def _gissue(c, slot, b):
            pltpu.async_copy(zel.at[islots.at[slot, 0]], zbs[b], gss[b])
            pltpu.async_copy(ert.at[islots.at[slot, 1]], ebs[b], ess[b])

        def _gwait(b):
            pltpu.make_async_copy(zel.at[islots.at[0, 0]], zbs[b],
                                  gss[b]).wait()
            pltpu.make_async_copy(ert.at[islots.at[0, 1]], ebs[b],
                                  ess[b]).wait()

        def _sissue(slot):
            pltpu.async_copy(ob, accum.at[islots.at[slot, 1]], ss, add=True)

        def _swait():
            pltpu.make_async_copy(ob, accum.at[islots.at[0, 1]], ss).wait()

        lax.fori_loop(0, CHUNK, _zero_ob, 0)
        for k in range(ZCH // NS + 1):
            ch = sid + NS * k
            if k * NS < ZCH:
                @pl.when(ch < ZCH)
                def _():
                    pltpu.sync_copy(ob, accum.at[pl.ds(ch * CHUNK, CHUNK)])
        plsc.subcore_barrier()

        for s in range(3):
            _icopy(cbase + s, s)
        _iwait(0)
        _gissue(cbase, 0, 0)
        _iwait(1)
        _gissue(cbase + 1, 1, 1)

        def _quad(k, _):
            for j in range(4):
                c = 4 * k + j
                b = j % 2
                _gwait(b)
                if j == 0:
                    @pl.when(k > 0)
                    def _():
                        _swait()
                else:
                    _swait()
                if 4 * 30 + j + 3 < NCHUNK:
                    _icopy(cbase + c + 3, (j + 3) % 4)
                else:
                    @pl.when(c + 3 < NCHUNK)
                    def _():
                        _icopy(cbase + c + 3, (j + 3) % 4)
                _compute(zbs[b], ebs[b])
                _sissue(j)
                if 4 * 30 + j + 2 < NCHUNK:
                    _iwait((j + 2) % 4)
                    _gissue(cbase + c + 2, (j + 2) % 4, b)
                else:
                    @pl.when(c + 2 < NCHUNK)
                    def _():
                        _iwait((j + 2) % 4)
                        _gissue(cbase + c + 2, (j + 2) % 4, b)
            return 0

        lax.fori_loop(0, NCHUNK // 4, _quad, 0)

        _gwait(0)
        _swait()
        _compute(zb0, eb0)
        _sissue(0)
        _swait()
        plsc.subcore_barrier()

        for k in range(ZCH // NS + 1):
            ch = sid + NS * k
            if k * NS < ZCH:
                @pl.when(ch < ZCH)
                def _():
                    pltpu.sync_copy(accum.at[pl.ds(ch * CHUNK, CHUNK)],
                                    out.at[cid, r, pl.ds(ch * CHUNK, CHUNK)])
        plsc.subcore_barrier()


_edge_call = functools.partial(
    pl.kernel,
    out_type=jax.ShapeDtypeStruct((NC, NREL, NNODE, ROWW), jnp.float32),
    mesh=plsc.VectorSubcoreMesh(core_axis_name="c", subcore_axis_name="s"),
    scratch_types=[
        pltpu.VMEM((4, 2, CHUNK), jnp.int32),
        pltpu.VMEM((CHUNK, ROWW), jnp.float32),
        pltpu.VMEM((CHUNK, ROWW), jnp.float32),
        pltpu.VMEM((CHUNK, ROWW), jnp.float32),
        pltpu.VMEM((CHUNK, ERW), jnp.float32),
        pltpu.VMEM((CHUNK, ERW), jnp.float32),
        pltpu.VMEM_SHARED((NNODE, ROWW), jnp.float32),
        pltpu.SemaphoreType.DMA,
        pltpu.SemaphoreType.DMA,
        pltpu.SemaphoreType.DMA,
        pltpu.SemaphoreType.DMA,
        pltpu.SemaphoreType.DMA,
        pltpu.SemaphoreType.DMA,
        pltpu.SemaphoreType.DMA,
        pltpu.SemaphoreType.DMA,
        pltpu.SemaphoreType.DMA,
    ],
    compiler_params=pltpu.CompilerParams(use_tc_tiling_on_sc=False,
                                         needs_layout_passes=False),
)(_edge_body)


def _fold(a):
    return (a[:, :, None] * jnp.eye(HEADS, dtype=a.dtype)[:, None, :]).reshape(HID, HEADS)


_SRC = (0, 0, 1, 1, 2)
_DST = (0, 1, 0, 1, 0)
BLK = 1000
NBLK = NNODE // BLK
_F32 = jnp.float32


def _dot(a, b):
    return jnp.dot(a, b, preferred_element_type=_F32)


def _k1_body(*refs):
    xd, xp, xs, wp, bp, wc, we = refs[:7]
    rd, rp, rs = refs[7:10]
    zels = refs[10:15]
    erts = refs[15:20]
    raws = []
    for t, x in enumerate((xd, xp, xs)):
        raws.append(_dot(x[...], wp[t]) + bp[t][None, :])
    rd[...] = raws[0]
    rp[...] = raws[1]
    rs[...] = raws[2]
    for r in range(NREL):
        zels[r][...] = _dot(raws[_SRC[r]], wc[r])
        erts[r][...] = _dot(raws[_DST[r]], we[r])


def _semantic(acc, sel, gb, w1, b1, w2):
    feats = []
    for r in range(NREL):
        a = acc[0, r] + acc[1, r]
        den = _dot(a[:, HID:HID + HEADS], sel[...])
        feats.append(a[:, :HID] / (den + 1e-9) + gb[r][None, :])
    hs = {}
    for d, rels in ((0, (0, 2, 4)), (1, (1, 3))):
        ws = [_dot(jnp.tanh(_dot(feats[r], w1[...]) + b1[...]), w2[...])
              for r in rels]
        m = functools.reduce(jnp.maximum, ws)
        es = [jnp.exp(w - m) for w in ws]
        dn = sum(es)
        hs[d] = sum((e_ / dn) * feats[r] for e_, r in zip(es, rels))
    return hs


def _k2_body(*refs):
    acc, sel, gb, w1, b1, w2, rs_blk, wc, we = refs[:9]
    zels = refs[9:14]
    erts = refs[14:19]
    hs = _semantic(acc, sel, gb, w1, b1, w2)
    hmap = (hs[0], hs[1], rs_blk[...])
    for r in range(NREL):
        zels[r][...] = _dot(hmap[_SRC[r]], wc[r])
        erts[r][...] = _dot(hmap[_DST[r]], we[r])


def _k3_body(*refs):
    acc, sel, gb, w1, b1, w2, hd, hp = refs
    hs = _semantic(acc, sel, gb, w1, b1, w2)
    hd[...] = hs[0]
    hp[...] = hs[1]


_row_spec = pl.BlockSpec((BLK, HID), lambda i: (i, 0))
_zel_spec = pl.BlockSpec((BLK, ROWW), lambda i: (i, 0))
_ert_spec = pl.BlockSpec((BLK, ERW), lambda i: (i, 0))
_zel_shape = jax.ShapeDtypeStruct((NNODE, ROWW), _F32)
_ert_shape = jax.ShapeDtypeStruct((NNODE, ERW), _F32)
_row_shape = jax.ShapeDtypeStruct((NNODE, HID), _F32)


def _whole(*shape):
    return pl.BlockSpec(shape, lambda i: tuple(0 for _ in shape))


_acc_spec = pl.BlockSpec((NC, NREL, BLK, ROWW), lambda i: (0, 0, i, 0))
_sem_specs = [_whole(8, HID), _whole(NREL, HID), _whole(HID, HID),
              _whole(1, HID), _whole(HID, 1)]

_k1 = pl.pallas_call(
    _k1_body,
    grid=(NBLK,),
    in_specs=[_row_spec, _row_spec, _row_spec, _whole(3, HID, HID),
              _whole(3, HID), _whole(NREL, HID, ROWW), _whole(NREL, HID, ERW)],
    out_specs=[_row_spec] * 3 + [_zel_spec] * NREL + [_ert_spec] * NREL,
    out_shape=[_row_shape] * 3 + [_zel_shape] * NREL + [_ert_shape] * NREL,
)

_k2 = pl.pallas_call(
    _k2_body,
    grid=(NBLK,),
    in_specs=[_acc_spec] + _sem_specs
    + [_row_spec, _whole(NREL, HID, ROWW), _whole(NREL, HID, ERW)],
    out_specs=[_zel_spec] * NREL + [_ert_spec] * NREL,
    out_shape=[_zel_shape] * NREL + [_ert_shape] * NREL,
)

_k3 = pl.pallas_call(
    _k3_body,
    grid=(NBLK,),
    in_specs=[_acc_spec] + _sem_specs,
    out_specs=[_row_spec, _row_spec],
    out_shape=[_row_shape, _row_shape],
)


def kernel(x_drug, x_protein, x_sideeffect, edge_drug2drug, edge_drug2protein,
           edge_protein2drug, edge_protein2protein, edge_sideeffect2drug, params):
    p = params

    wp = jnp.stack([p['proj_W_drug'], p['proj_W_protein'], p['proj_W_sideeffect']])
    bp = jnp.stack([p['proj_b_drug'], p['proj_b_protein'], p['proj_b_sideeffect']])
    sel = (jnp.arange(HID, dtype=jnp.int32) // OUTF ==
           jnp.arange(HEADS, dtype=jnp.int32)[:, None]).astype(_F32)
    zpad8 = jnp.zeros((HID, 8), _F32)
    wcs, wes, gbs = [], [], []
    for l in range(2):
        wc_l, we_l, gb_l = [], [], []
        for rel, s, d in _RELS:
            W = p['gat_W_%d_%s' % (l, rel)]
            wc_l.append(jnp.concatenate(
                [W, W @ _fold(p['gat_al_%d_%s' % (l, rel)]), zpad8], axis=1))
            we_l.append(jnp.concatenate(
                [W @ _fold(p['gat_ar_%d_%s' % (l, rel)]), zpad8], axis=1))
            gb_l.append(p['gat_b_%d_%s' % (l, rel)])
        wcs.append(jnp.stack(wc_l))
        wes.append(jnp.stack(we_l))
        gbs.append(jnp.stack(gb_l))
    sems = [(sel, gbs[l], p['sem_W1_%d' % l], p['sem_b1_%d' % l].reshape(1, HID),
             p['sem_W2_%d' % l]) for l in range(2)]
    e3s = [edges.astype(jnp.int32).reshape(2, E // CHUNK, CHUNK).transpose(1, 0, 2)
           for edges in (edge_drug2drug, edge_drug2protein, edge_protein2drug,
                         edge_protein2protein, edge_sideeffect2drug)]

    outs1 = _k1(x_drug, x_protein, x_sideeffect, wp, bp, wcs[0], wes[0])
    raw_side = outs1[2]
    acc0 = _edge_call(*outs1[3:8], *outs1[8:13], *e3s)
    outs2 = _k2(acc0, *sems[0], raw_side, wcs[1], wes[1])
    acc1 = _edge_call(*outs2[0:5], *outs2[5:10], *e3s)
    h_drug, h_prot = _k3(acc1, *sems[1])
    return h_drug, h_prot, raw_side
